# Initial kernel scaffold; baseline (speedup 1.0000x reference)
import jax
import jax.numpy as jnp
from jax.experimental import pallas as pl


def kernel(uEmbeds, iEmbeds, edge_index, edge_weight):
    raise NotImplementedError("write your pallas kernel here")


# speedup vs baseline: 6.7194x; 6.7194x over previous
import functools

import jax
import jax.numpy as jnp
from jax import lax
from jax.experimental import pallas as pl
from jax.experimental.pallas import tpu as pltpu
from jax.experimental.pallas import tpu_sc as plsc

N_NODES = 10000
LATDIM = 128
N_EDGES = 320000

NC = 2
NS = 16
NW = NC * NS

B = 128
EDGES_PER_TILE = -(-N_EDGES // (NW * B)) * B
CHUNKS = EDGES_PER_TILE // B
E_PAD = NW * EDGES_PER_TILE
ROWS_PER_TILE = 632
N_PAD = NS * ROWS_PER_TILE


def _spmm_body(x_hbm, src_hbm, dst_hbm, w_hbm, out_hbm,
               src_v, dst_v, w_v, rows_v, part_sh, sem):
    c = lax.axis_index("c")
    s = lax.axis_index("s")
    wid = c * NS + s

    def zrow(r, _):
        for f in range(LATDIM // 16):
            rows_v[r, pl.ds(f * 16, 16)] = jnp.zeros((16,), jnp.float32)
        return _
    lax.fori_loop(0, B, zrow, None)
    for st, sz in ((0, B), (B, B), (2 * B, B), (3 * B, B), (4 * B, 120)):
        pltpu.sync_copy(rows_v.at[pl.ds(0, sz)],
                        part_sh.at[pl.ds(s * ROWS_PER_TILE + st, sz)])

    pltpu.sync_copy(src_hbm.at[wid], src_v)
    pltpu.sync_copy(dst_hbm.at[wid], dst_v)
    pltpu.sync_copy(w_hbm.at[wid], w_v)

    plsc.subcore_barrier()

    bcast_dnums = lax.GatherDimensionNumbers(
        offset_dims=(), collapsed_slice_dims=(0,), start_index_map=(0,))

    def bcast_lane(vec, l):
        idx = jnp.full((16, 1), l, jnp.int32)
        return lax.gather(vec, idx, bcast_dnums, slice_sizes=(1,),
                          mode=lax.GatherScatterMode.PROMISE_IN_BOUNDS)

    def scale_group(j):
        def body(g, _):
            w16 = w_v[j, pl.ds(g * 16, 16)]
            for l in range(16):
                wl = bcast_lane(w16, l)
                r = g * 16 + l
                for f in range(LATDIM // 16):
                    sl = pl.ds(f * 16, 16)
                    rows_v[r, sl] = rows_v[r, sl] * wl
            return _
        lax.fori_loop(0, B // 16, body, None)

    def edge_chunk(j, _):
        pltpu.async_copy(x_hbm.at[src_v.at[j]], rows_v, sem).wait()
        scale_group(j)
        pltpu.sync_copy(rows_v, part_sh.at[dst_v.at[j]], add=True)
        return _
    lax.fori_loop(0, CHUNKS, edge_chunk, None)

    plsc.subcore_barrier()

    rs = s * ROWS_PER_TILE
    pltpu.sync_copy(part_sh.at[pl.ds(rs, ROWS_PER_TILE)],
                    out_hbm.at[c, pl.ds(rs, ROWS_PER_TILE)])


_spmm = functools.partial(
    pl.kernel,
    out_type=jax.ShapeDtypeStruct((NC, N_PAD, LATDIM), jnp.float32),
    mesh=plsc.VectorSubcoreMesh(core_axis_name="c", subcore_axis_name="s"),
    scratch_types=[
        pltpu.VMEM((CHUNKS, B), jnp.int32),
        pltpu.VMEM((CHUNKS, B), jnp.int32),
        pltpu.VMEM((CHUNKS, B), jnp.float32),
        pltpu.VMEM((B, LATDIM), jnp.float32),
        pltpu.VMEM_SHARED((N_PAD, LATDIM), jnp.float32),
        pltpu.SemaphoreType.DMA,
    ],
)(_spmm_body)


def _merge_body(p_ref, base_ref, x_ref, t_ref):
    x = p_ref[0] + p_ref[1]
    x_ref[...] = x
    t_ref[...] = base_ref[...] + x


_MR = 400


def _merge(parts, base):
    grid = N_NODES // _MR
    return pl.pallas_call(
        _merge_body,
        grid=(grid,),
        in_specs=[
            pl.BlockSpec((NC, _MR, LATDIM), lambda i: (0, i, 0)),
            pl.BlockSpec((_MR, LATDIM), lambda i: (i, 0)),
        ],
        out_specs=[
            pl.BlockSpec((_MR, LATDIM), lambda i: (i, 0)),
            pl.BlockSpec((_MR, LATDIM), lambda i: (i, 0)),
        ],
        out_shape=[
            jax.ShapeDtypeStruct((N_NODES, LATDIM), jnp.float32),
            jax.ShapeDtypeStruct((N_NODES, LATDIM), jnp.float32),
        ],
    )(parts, base)


def kernel(uEmbeds, iEmbeds, edge_index, edge_weight):
    ini = jnp.concatenate([uEmbeds, iEmbeds], axis=0)
    src = edge_index[0].astype(jnp.int32)
    dst = edge_index[1].astype(jnp.int32)
    w = edge_weight.astype(jnp.float32)

    pad = E_PAD - N_EDGES
    pad_idx = (jnp.arange(pad, dtype=jnp.int32) * 37) % N_NODES
    src_p = jnp.concatenate([src, pad_idx]).reshape(NW, CHUNKS, B)
    dst_p = jnp.concatenate([dst, pad_idx]).reshape(NW, CHUNKS, B)
    w_p = jnp.concatenate([w, jnp.zeros((pad,), jnp.float32)]).reshape(
        NW, CHUNKS, B)

    parts1 = _spmm(ini, src_p, dst_p, w_p)
    x1, total1 = _merge(parts1, ini)
    parts2 = _spmm(x1, src_p, dst_p, w_p)
    _, total2 = _merge(parts2, total1)
    return total2
